# baseline (speedup 1.0000x reference)
import functools

import jax
import jax.numpy as jnp
from jax import lax
from jax.experimental import pallas as pl
from jax.experimental.pallas import tpu as pltpu
import jax.experimental.pallas.tpu_sc as plsc

HID = 128
HEADS = 8
HC = HID * HEADS
NREQ = 10000
NVEH = 10000
NPAD = 10240
MBLK = 1152
EPAD = {"rr": 102400, "vr": 61440, "rv": 61440}
EREAL = {"rr": 100000, "vr": 60000, "rv": 60000}
BN = 512
SC_ROWS = 10112


def _mm_std(x, w, b=None):
    np_, k = x.shape
    m = w.shape[1]

    def body(x_ref, w_ref, *rest):
        if b is not None:
            b_ref, o_ref = rest
            o_ref[...] = (
                jnp.dot(x_ref[...], w_ref[...],
                        preferred_element_type=jnp.float32) + b_ref[...])
        else:
            (o_ref,) = rest
            o_ref[...] = jnp.dot(x_ref[...], w_ref[...],
                                 preferred_element_type=jnp.float32)

    in_specs = [
        pl.BlockSpec((BN, k), lambda i: (i, 0)),
        pl.BlockSpec((k, m), lambda i: (0, 0)),
    ]
    args = [x, w]
    if b is not None:
        in_specs.append(pl.BlockSpec((1, m), lambda i: (0, 0)))
        args.append(b.reshape(1, m))
    return pl.pallas_call(
        body,
        grid=(np_ // BN,),
        in_specs=in_specs,
        out_specs=pl.BlockSpec((BN, m), lambda i: (i, 0)),
        out_shape=jax.ShapeDtypeStruct((np_, m), jnp.float32),
    )(*args)


def _mm_hm(x, w):
    np_ = x.shape[1]
    m = w.shape[2]

    def body(x_ref, w_ref, o_ref):
        h = pl.program_id(1)

        @pl.when(h == 0)
        def _():
            o_ref[...] = jnp.zeros_like(o_ref)

        o_ref[...] += jnp.dot(x_ref[0], w_ref[0],
                              preferred_element_type=jnp.float32)

    return pl.pallas_call(
        body,
        grid=(np_ // BN, HEADS),
        in_specs=[
            pl.BlockSpec((1, BN, HID), lambda i, h: (h, i, 0)),
            pl.BlockSpec((1, HID, m), lambda i, h: (h, 0, 0)),
        ],
        out_specs=pl.BlockSpec((BN, m), lambda i, h: (i, 0)),
        out_shape=jax.ShapeDtypeStruct((np_, m), jnp.float32),
        compiler_params=pltpu.CompilerParams(
            dimension_semantics=("parallel", "arbitrary")),
    )(x, w)


def _combine(raws, dens, bs, resid):
    k = len(raws)

    def body(*refs):
        o_ref = refs[-1]
        h = pl.program_id(0)
        mask = (lax.broadcasted_iota(jnp.int32, (1, 128), 1) == h)
        acc = jnp.zeros((BN, HID), jnp.float32)
        idx = 0
        for j in range(k):
            raw_ref = refs[idx]; den_ref = refs[idx + 1]; b_ref = refs[idx + 2]
            idx += 3
            d = den_ref[0] + den_ref[1]
            d_h = jnp.sum(jnp.where(mask, d, 0.0), axis=1, keepdims=True)
            acc += raw_ref[0] / (d_h + 1e-16) + b_ref[0]
        acc = acc / float(k)
        if resid is not None:
            acc += refs[idx][0]
        o_ref[0] = jnp.where(acc > 0, acc, 0.01 * acc)

    in_specs = []
    args = []
    for j in range(k):
        in_specs += [
            pl.BlockSpec((1, BN, HID), lambda h, i: (h, i, 0)),
            pl.BlockSpec((2, BN, 128), lambda h, i: (0, i, 0)),
            pl.BlockSpec((1, 1, HID), lambda h, i: (h, 0, 0)),
        ]
        args += [raws[j], dens[j], bs[j].reshape(HEADS, 1, HID)]
    if resid is not None:
        in_specs.append(pl.BlockSpec((1, BN, HID), lambda h, i: (h, i, 0)))
        args.append(resid)
    return pl.pallas_call(
        body,
        grid=(HEADS, NPAD // BN),
        in_specs=in_specs,
        out_specs=pl.BlockSpec((1, BN, HID), lambda h, i: (h, i, 0)),
        out_shape=jax.ShapeDtypeStruct((HEADS, NPAD, HID), jnp.float32),
    )(*args)


def _lanesum(x, b2):
    ep = x.shape[0]

    def body(x_ref, b_ref, o_ref):
        o_ref[...] = jnp.sum(x_ref[...], axis=1, keepdims=True) + b_ref[0, 0]

    return pl.pallas_call(
        body,
        grid=(ep // BN,),
        in_specs=[pl.BlockSpec((BN, 16), lambda i: (i, 0)),
                  pl.BlockSpec((1, 1), lambda i: (0, 0),
                               memory_space=pltpu.SMEM)],
        out_specs=pl.BlockSpec((BN, 1), lambda i: (i, 0)),
        out_shape=jax.ShapeDtypeStruct((ep, 1), jnp.float32),
    )(x, b2.reshape(1, 1))


def _pack_rec(s2, d2):
    nrows = s2.shape[0]
    bn = 80

    def body(s_ref, d_ref, o_ref):
        s = s_ref[...]
        d = d_ref[...]
        o_ref[:, 0, :] = s * 9
        o_ref[:, 1, :] = d * 9
        o_ref[:, 2, :] = d
        o_ref[:, 3, :] = s

    return pl.pallas_call(
        body,
        grid=(nrows // bn,),
        in_specs=[pl.BlockSpec((bn, 128), lambda i: (i, 0)),
                  pl.BlockSpec((bn, 128), lambda i: (i, 0))],
        out_specs=pl.BlockSpec((bn, 4, 128), lambda i: (i, 0, 0)),
        out_shape=jax.ShapeDtypeStruct((nrows, 4, 128), jnp.int32),
    )(s2, d2)


NC = 2
NS = 16


def _sc_mesh():
    return plsc.VectorSubcoreMesh(core_axis_name="c", subcore_axis_name="s")


def _edge_exp(hst, hdt, rec):
    nrows = rec.shape[0]
    epad = nrows * 128
    tile_rows = nrows // (NC * NS)
    full, rem = divmod(tile_rows, 2)

    @functools.partial(
        pl.kernel,
        out_type=jax.ShapeDtypeStruct((epad, 16), jnp.float32),
        mesh=_sc_mesh(),
        scratch_types=[
            pltpu.VMEM((2, 4, 128), jnp.int32),
            pltpu.VMEM((2, 1, 128), jnp.int32),
            pltpu.VMEM((2, 1, 128), jnp.int32),
            pltpu.VMEM((256, 128), jnp.float32),
            pltpu.VMEM((256, 128), jnp.float32),
            pltpu.VMEM((256, 16), jnp.float32),
            pltpu.SemaphoreType.DMA,
        ])
    def k(hst_h, hdt_h, rec_h, ex_h,
          recv, gsx, gdx, asv, adv, exv16, sem):
        c = lax.axis_index("c")
        s = lax.axis_index("s")
        w = s * NC + c
        row0 = w * tile_rows

        def do(rbase, nsub):
            pltpu.sync_copy(rec_h.at[pl.ds(row0 + rbase, nsub)],
                            recv.at[pl.ds(0, nsub)])
            for j in range(nsub):
                for t in range(8):
                    gsx[j, 0, pl.ds(t * 16, 16)] = (
                        recv[j, 0, pl.ds(t * 16, 16)] + 8)
                    gdx[j, 0, pl.ds(t * 16, 16)] = (
                        recv[j, 1, pl.ds(t * 16, 16)] + 8)
            cps = []
            for j in range(nsub):
                cps.append(pltpu.async_copy(
                    hst_h.at[gsx.at[j, 0]], asv.at[pl.ds(j * 128, 128)], sem))
                cps.append(pltpu.async_copy(
                    hdt_h.at[gdx.at[j, 0]], adv.at[pl.ds(j * 128, 128)], sem))
            for cp in cps:
                cp.wait()

            def comp(e, _):
                a = asv[e, pl.ds(0, 16)] + adv[e, pl.ds(0, 16)]
                al = jnp.where(a > 0, a, 0.2 * a)
                exv16[e, :] = jnp.exp(al)
                return 0
            lax.fori_loop(0, nsub * 128, comp, 0)
            pltpu.sync_copy(exv16.at[pl.ds(0, nsub * 128)],
                            ex_h.at[pl.ds((row0 + rbase) * 128, nsub * 128)])

        if full:
            lax.fori_loop(0, full, lambda t, _: (do(t * 2, 2), 0)[1], 0)
        if rem:
            do(full * 2, rem)

    return k(hst, hdt, rec)


def _edge_message(hst, rec, ex):
    nrows = rec.shape[0]
    tile_rows = nrows // NS
    tile_rows_d = nrows // (NC * NS)

    @functools.partial(
        pl.kernel,
        out_type=(jax.ShapeDtypeStruct((HEADS, NPAD, HID), jnp.float32),
                  jax.ShapeDtypeStruct((NC, NPAD, 128), jnp.float32)),
        mesh=_sc_mesh(),
        scratch_types=[
            pltpu.VMEM((1, 4, 128), jnp.int32),
            pltpu.VMEM((1, 1, 128), jnp.int32),
            pltpu.VMEM((128, 16), jnp.float32),
            pltpu.VMEM((128, 128), jnp.float32),
            pltpu.VMEM_SHARED((SC_ROWS, HID), jnp.float32),
            pltpu.SemaphoreType.DMA,
        ])
    def k(hst_h, rec_h, ex_h, out_h, den_h,
          recv, gidx, exv, rows, out_sh, sem):
        c = lax.axis_index("c")
        s = lax.axis_index("s")

        def zero_rows():
            def fz(i, _):
                for t in range(8):
                    rows[i, pl.ds(t * 16, 16)] = jnp.zeros((16,), jnp.float32)
                return 0
            lax.fori_loop(0, 128, fz, 0)

        def zero_out_sh():
            zero_rows()
            for z in range(4):
                pltpu.sync_copy(rows,
                                out_sh.at[pl.ds(s * 632 + z * 128, 128)])
            pltpu.sync_copy(rows.at[pl.ds(0, 120)],
                            out_sh.at[pl.ds(s * 632 + 512, 120)])

        gdn = lax.GatherDimensionNumbers(
            offset_dims=(), collapsed_slice_dims=(0,), start_index_map=(0,))

        for h in range(4):
            head = c * 4 + h
            hidx = jnp.full((16, 1), head, jnp.int32)
            row0 = s * tile_rows
            zero_out_sh()
            plsc.subcore_barrier()

            def do(r, _):
                pltpu.sync_copy(rec_h.at[pl.ds(row0 + r, 1)], recv)
                for t in range(8):
                    gidx[0, 0, pl.ds(t * 16, 16)] = (
                        recv[0, 0, pl.ds(t * 16, 16)] + head)
                cp = pltpu.async_copy(hst_h.at[gidx.at[0, 0]], rows, sem)
                pltpu.sync_copy(ex_h.at[pl.ds((row0 + r) * 128, 128)], exv)
                cp.wait()

                def scale(e, _):
                    ev = exv[e, :]
                    wv = lax.gather(
                        ev, hidx, gdn, slice_sizes=(1,),
                        mode=lax.GatherScatterMode.PROMISE_IN_BOUNDS)
                    for t in range(8):
                        rows[e, pl.ds(t * 16, 16)] = (
                            rows[e, pl.ds(t * 16, 16)] * wv)
                    return 0
                lax.fori_loop(0, 128, scale, 0)
                pltpu.sync_copy(rows, out_sh.at[recv.at[0, 2]], add=True)
                return 0

            lax.fori_loop(0, tile_rows, do, 0)
            plsc.subcore_barrier()
            pltpu.sync_copy(out_sh.at[pl.ds(s * 632, 632)],
                            out_h.at[head, pl.ds(s * 632, 632)])
            plsc.subcore_barrier()

        zero_out_sh()
        plsc.subcore_barrier()
        row0_d = (s * NC + c) * tile_rows_d

        def do_den(r, _):
            pltpu.sync_copy(rec_h.at[pl.ds(row0_d + r, 1)], recv)
            pltpu.sync_copy(ex_h.at[pl.ds((row0_d + r) * 128, 128)], exv)

            def cpex(e, _):
                rows[e, pl.ds(0, 16)] = exv[e, :]
                return 0
            lax.fori_loop(0, 128, cpex, 0)
            pltpu.sync_copy(rows, out_sh.at[recv.at[0, 2]], add=True)
            return 0

        lax.fori_loop(0, tile_rows_d, do_den, 0)
        plsc.subcore_barrier()
        pltpu.sync_copy(out_sh.at[pl.ds(s * 632, 632)],
                        den_h.at[c, pl.ds(s * 632, 632)])

    return k(hst, rec, ex)


def _edge_final(ptab_s, ptab_d, rec, eap, w2):
    nrows = rec.shape[0]
    epad = nrows * 128
    tile_rows = nrows // (NC * NS)

    @functools.partial(
        pl.kernel,
        out_type=jax.ShapeDtypeStruct((epad, 16), jnp.float32),
        mesh=_sc_mesh(),
        scratch_types=[
            pltpu.VMEM((1, 4, 128), jnp.int32),
            pltpu.VMEM((1, 1, 128), jnp.int32),
            pltpu.VMEM((1, 1, 128), jnp.int32),
            pltpu.VMEM((128, 128), jnp.float32),
            pltpu.VMEM((128, 128), jnp.float32),
            pltpu.VMEM((128, 128), jnp.float32),
            pltpu.VMEM((128,), jnp.float32),
            pltpu.VMEM((128, 16), jnp.float32),
            pltpu.SemaphoreType.DMA,
        ])
    def k(ps_h, pd_h, rec_h, eap_h, w2_h, lanes_h,
          recv, gs, gd, psv, pdv, eav, w2v, lanev, sem):
        c = lax.axis_index("c")
        s = lax.axis_index("s")
        w = s * NC + c
        row0 = w * tile_rows
        pltpu.sync_copy(w2_h, w2v)

        def do(r, _):
            pltpu.sync_copy(rec_h.at[pl.ds(row0 + r, 1)], recv)
            for t in range(8):
                gs[0, 0, pl.ds(t * 16, 16)] = recv[0, 3, pl.ds(t * 16, 16)] * 2
                gd[0, 0, pl.ds(t * 16, 16)] = (
                    recv[0, 2, pl.ds(t * 16, 16)] * 2 + 1)
            cp1 = pltpu.async_copy(ps_h.at[gs.at[0, 0]], psv, sem)
            cp2 = pltpu.async_copy(pd_h.at[gd.at[0, 0]], pdv, sem)
            pltpu.sync_copy(eap_h.at[pl.ds((row0 + r) * 128, 128)], eav)
            cp1.wait()
            cp2.wait()

            def comp(e, _):
                u = jnp.zeros((16,), jnp.float32)
                for t in range(8):
                    v = (psv[e, pl.ds(t * 16, 16)]
                         + pdv[e, pl.ds(t * 16, 16)]
                         + eav[e, pl.ds(t * 16, 16)])
                    v = jnp.where(v > 0, v, 0.01 * v)
                    u = u + v * w2v[pl.ds(t * 16, 16)]
                lanev[e, :] = u
                return 0
            lax.fori_loop(0, 128, comp, 0)
            pltpu.sync_copy(lanev,
                            lanes_h.at[pl.ds((row0 + r) * 128, 128)])
            return 0

        lax.fori_loop(0, tile_rows, do, 0)

    return k(ptab_s, ptab_d, rec, eap, w2)


def _alcols(w, a):
    wr = w.reshape(w.shape[0], HEADS, HID)
    al = jnp.einsum("dhj,hj->dh", wr, a)
    return jnp.pad(al, ((0, 0), (0, 8)))


def _wcat(w, a):
    return jnp.concatenate(
        [w, _alcols(w, a), jnp.zeros((w.shape[0], MBLK - HC - 16), jnp.float32)],
        axis=1)


def _pad_rows(x, n):
    return jnp.pad(x, ((0, n - x.shape[0]), (0, 0)))


def kernel(x_request, x_vehicle,
           edge_index_rr, edge_attr_rr,
           edge_index_vr, edge_attr_vr,
           edge_index_rv, edge_attr_rv,
           Wsrc_rr1, Wdst_rr1, asrc_rr1, adst_rr1, b_rr1,
           Wsrc_vr1, Wdst_vr1, asrc_vr1, adst_vr1, b_vr1,
           Wsrc_rv1, Wdst_rv1, asrc_rv1, adst_rv1, b_rv1,
           Wsrc_rr2, Wdst_rr2, asrc_rr2, adst_rr2, b_rr2,
           Wsrc_vr2, Wdst_vr2, asrc_vr2, adst_vr2, b_vr2,
           Wsrc_rv2, Wdst_rv2, asrc_rv2, adst_rv2, b_rv2,
           Wep_rr, bep_rr, Wep_vr, bep_vr, Wep_rv, bep_rv,
           W1, b1, W2, b2):
    p = dict(locals())
    etypes = [("rr", "request", "request"),
              ("vr", "vehicle", "request"),
              ("rv", "request", "vehicle")]

    src = {}
    dst = {}
    for et in EPAD:
        ei = p[f"edge_index_{et}"]
        e = EREAL[et]
        s = jnp.full((EPAD[et],), 10000, jnp.int32).at[:e].set(ei[0])
        d = jnp.full((EPAD[et],), 10000, jnp.int32).at[:e].set(ei[1])
        src[et] = _pack_rec(s.reshape(-1, 128), d.reshape(-1, 128))

    x = {"request": _pad_rows(x_request, NPAD),
         "vehicle": _pad_rows(x_vehicle, NPAD)}

    for l in (1, 2):
        wc = {et: {"s": _wcat(p[f"Wsrc_{et}{l}"], p[f"asrc_{et}{l}"]),
                   "d": _wcat(p[f"Wdst_{et}{l}"], p[f"adst_{et}{l}"])}
              for et in EPAD}
        w_req = jnp.concatenate(
            [wc["rr"]["s"], wc["rr"]["d"], wc["vr"]["d"], wc["rv"]["s"]], axis=1)
        w_veh = jnp.concatenate([wc["vr"]["s"], wc["rv"]["d"]], axis=1)
        if l == 1:
            out_req = _mm_std(x["request"], w_req)
            out_veh = _mm_std(x["vehicle"], w_veh)
        else:
            out_req = _mm_hm(x["request"], w_req.reshape(HEADS, HID, -1))
            out_veh = _mm_hm(x["vehicle"], w_veh.reshape(HEADS, HID, -1))

        def _block(o, i):
            blk = o[:, i * MBLK:(i + 1) * MBLK]
            return blk.reshape(NPAD * 9, HID)

        hst = {"rr": _block(out_req, 0), "vr": _block(out_veh, 0),
               "rv": _block(out_req, 3)}
        hdt = {"rr": _block(out_req, 1), "vr": _block(out_req, 2),
               "rv": _block(out_veh, 1)}

        raws = {"request": [], "vehicle": []}
        dens = {"request": [], "vehicle": []}
        bs = {"request": [], "vehicle": []}
        for et, st, dt in etypes:
            ex = _edge_exp(hst[et], hdt[et], src[et])
            raw, den = _edge_message(hst[et], src[et], ex)
            raws[dt].append(raw)
            dens[dt].append(den)
            bs[dt].append(p[f"b_{et}{l}"].reshape(HEADS, HID))

        newx = {}
        for dt in ("request", "vehicle"):
            resid = x[dt] if l == 2 else None
            newx[dt] = _combine(raws[dt], dens[dt], bs[dt], resid)
        x = newx

    W1s, W1e, W1t = W1[:HC], W1[HC:HC + HID], W1[HC + HID:]
    wst = jnp.concatenate([W1s.reshape(HEADS, HID, HID),
                           W1t.reshape(HEADS, HID, HID)], axis=2)
    ptab = {"request": _mm_hm(x["request"], wst).reshape(NPAD * 2, HID),
            "vehicle": _mm_hm(x["vehicle"], wst).reshape(NPAD * 2, HID)}
    w2 = W2.reshape(HID)

    outs = []
    for et, st, dt in etypes:
        wea = p[f"Wep_{et}"] @ W1e
        bea = p[f"bep_{et}"] @ W1e + b1
        ea = jnp.pad(p[f"edge_attr_{et}"], ((0, EPAD[et] - EREAL[et]), (0, 0)))
        eap = _mm_std(ea, wea, bea)
        lanes = _edge_final(ptab[st], ptab[dt], src[et], eap, w2)
        res = _lanesum(lanes, b2[0])
        outs.append(res[:EREAL[et]])
    return jnp.concatenate(outs, axis=0)


# speedup vs baseline: 1.0829x; 1.0829x over previous
import functools

import jax
import jax.numpy as jnp
from jax import lax
from jax.experimental import pallas as pl
from jax.experimental.pallas import tpu as pltpu
import jax.experimental.pallas.tpu_sc as plsc

HID = 128
HEADS = 8
HC = HID * HEADS
NREQ = 10000
NVEH = 10000
NPAD = 10240
MBLK = 1152
EPAD = {"rr": 102400, "vr": 61440, "rv": 61440}
EREAL = {"rr": 100000, "vr": 60000, "rv": 60000}
BN = 512
SC_ROWS = 10112


def _mm_std(x, w, b=None):
    np_, k = x.shape
    m = w.shape[1]

    def body(x_ref, w_ref, *rest):
        if b is not None:
            b_ref, o_ref = rest
            o_ref[...] = (
                jnp.dot(x_ref[...], w_ref[...],
                        preferred_element_type=jnp.float32) + b_ref[...])
        else:
            (o_ref,) = rest
            o_ref[...] = jnp.dot(x_ref[...], w_ref[...],
                                 preferred_element_type=jnp.float32)

    in_specs = [
        pl.BlockSpec((BN, k), lambda i: (i, 0)),
        pl.BlockSpec((k, m), lambda i: (0, 0)),
    ]
    args = [x, w]
    if b is not None:
        in_specs.append(pl.BlockSpec((1, m), lambda i: (0, 0)))
        args.append(b.reshape(1, m))
    return pl.pallas_call(
        body,
        grid=(np_ // BN,),
        in_specs=in_specs,
        out_specs=pl.BlockSpec((BN, m), lambda i: (i, 0)),
        out_shape=jax.ShapeDtypeStruct((np_, m), jnp.float32),
    )(*args)


def _mm_hm(x, w):
    np_ = x.shape[1]
    m = w.shape[2]

    def body(x_ref, w_ref, o_ref):
        h = pl.program_id(1)

        @pl.when(h == 0)
        def _():
            o_ref[...] = jnp.zeros_like(o_ref)

        o_ref[...] += jnp.dot(x_ref[0], w_ref[0],
                              preferred_element_type=jnp.float32)

    return pl.pallas_call(
        body,
        grid=(np_ // BN, HEADS),
        in_specs=[
            pl.BlockSpec((1, BN, HID), lambda i, h: (h, i, 0)),
            pl.BlockSpec((1, HID, m), lambda i, h: (h, 0, 0)),
        ],
        out_specs=pl.BlockSpec((BN, m), lambda i, h: (i, 0)),
        out_shape=jax.ShapeDtypeStruct((np_, m), jnp.float32),
        compiler_params=pltpu.CompilerParams(
            dimension_semantics=("parallel", "arbitrary")),
    )(x, w)


def _combine(raws, dens, bs, resid):
    k = len(raws)

    def body(*refs):
        o_ref = refs[-1]
        h = pl.program_id(0)
        mask = (lax.broadcasted_iota(jnp.int32, (1, 128), 1) == h)
        acc = jnp.zeros((BN, HID), jnp.float32)
        idx = 0
        for j in range(k):
            raw_ref = refs[idx]; den_ref = refs[idx + 1]; b_ref = refs[idx + 2]
            idx += 3
            d = den_ref[0] + den_ref[1]
            d_h = jnp.sum(jnp.where(mask, d, 0.0), axis=1, keepdims=True)
            acc += raw_ref[0] / (d_h + 1e-16) + b_ref[0]
        acc = acc / float(k)
        if resid is not None:
            acc += refs[idx][0]
        o_ref[0] = jnp.where(acc > 0, acc, 0.01 * acc)

    in_specs = []
    args = []
    for j in range(k):
        in_specs += [
            pl.BlockSpec((1, BN, HID), lambda h, i: (h, i, 0)),
            pl.BlockSpec((2, BN, 128), lambda h, i: (0, i, 0)),
            pl.BlockSpec((1, 1, HID), lambda h, i: (h, 0, 0)),
        ]
        args += [raws[j], dens[j], bs[j].reshape(HEADS, 1, HID)]
    if resid is not None:
        in_specs.append(pl.BlockSpec((1, BN, HID), lambda h, i: (h, i, 0)))
        args.append(resid)
    return pl.pallas_call(
        body,
        grid=(HEADS, NPAD // BN),
        in_specs=in_specs,
        out_specs=pl.BlockSpec((1, BN, HID), lambda h, i: (h, i, 0)),
        out_shape=jax.ShapeDtypeStruct((HEADS, NPAD, HID), jnp.float32),
    )(*args)


def _lanesum(x, b2):
    ep = x.shape[0]

    def body(x_ref, b_ref, o_ref):
        o_ref[...] = jnp.sum(x_ref[...], axis=1, keepdims=True) + b_ref[0, 0]

    return pl.pallas_call(
        body,
        grid=(ep // BN,),
        in_specs=[pl.BlockSpec((BN, 16), lambda i: (i, 0)),
                  pl.BlockSpec((1, 1), lambda i: (0, 0),
                               memory_space=pltpu.SMEM)],
        out_specs=pl.BlockSpec((BN, 1), lambda i: (i, 0)),
        out_shape=jax.ShapeDtypeStruct((ep, 1), jnp.float32),
    )(x, b2.reshape(1, 1))


def _pack_rec(s2, d2):
    nrows = s2.shape[0]
    bn = 80

    def body(s_ref, d_ref, o_ref):
        s = s_ref[...]
        d = d_ref[...]
        o_ref[:, 0, :] = s
        o_ref[:, 1, :] = d

    return pl.pallas_call(
        body,
        grid=(nrows // bn,),
        in_specs=[pl.BlockSpec((bn, 128), lambda i: (i, 0)),
                  pl.BlockSpec((bn, 128), lambda i: (i, 0))],
        out_specs=pl.BlockSpec((bn, 2, 128), lambda i: (i, 0, 0)),
        out_shape=jax.ShapeDtypeStruct((nrows, 2, 128), jnp.int32),
    )(s2, d2)


NC = 2
NS = 16


def _sc_mesh():
    return plsc.VectorSubcoreMesh(core_axis_name="c", subcore_axis_name="s")


def _edge_exp(tab_s, ss, os_, tab_d, sd, od, rec):
    nrows = rec.shape[0]
    epad = nrows * 128
    tile_rows = nrows // (NC * NS)
    full, rem = divmod(tile_rows, 2)

    @functools.partial(
        pl.kernel,
        out_type=jax.ShapeDtypeStruct((epad, 16), jnp.float32),
        mesh=_sc_mesh(),
        scratch_types=[
            pltpu.VMEM((2, 2, 128), jnp.int32),
            pltpu.VMEM((2, 1, 128), jnp.int32),
            pltpu.VMEM((2, 1, 128), jnp.int32),
            pltpu.VMEM((256, 128), jnp.float32),
            pltpu.VMEM((256, 128), jnp.float32),
            pltpu.VMEM((256, 16), jnp.float32),
            pltpu.SemaphoreType.DMA,
        ])
    def k(ts_h, td_h, rec_h, ex_h,
          recv, gsx, gdx, asv, adv, exv16, sem):
        c = lax.axis_index("c")
        s = lax.axis_index("s")
        w = s * NC + c
        row0 = w * tile_rows

        def do(rbase, nsub):
            pltpu.sync_copy(rec_h.at[pl.ds(row0 + rbase, nsub)],
                            recv.at[pl.ds(0, nsub)])
            for j in range(nsub):
                for t in range(8):
                    gsx[j, 0, pl.ds(t * 16, 16)] = (
                        recv[j, 0, pl.ds(t * 16, 16)] * ss + os_)
                    gdx[j, 0, pl.ds(t * 16, 16)] = (
                        recv[j, 1, pl.ds(t * 16, 16)] * sd + od)
            cps = []
            for j in range(nsub):
                cps.append(pltpu.async_copy(
                    ts_h.at[gsx.at[j, 0]], asv.at[pl.ds(j * 128, 128)], sem))
                cps.append(pltpu.async_copy(
                    td_h.at[gdx.at[j, 0]], adv.at[pl.ds(j * 128, 128)], sem))
            for cp in cps:
                cp.wait()

            def comp(e, _):
                a = asv[e, pl.ds(0, 16)] + adv[e, pl.ds(0, 16)]
                al = jnp.where(a > 0, a, 0.2 * a)
                exv16[e, :] = jnp.exp(al)
                return 0
            lax.fori_loop(0, nsub * 128, comp, 0)
            pltpu.sync_copy(exv16.at[pl.ds(0, nsub * 128)],
                            ex_h.at[pl.ds((row0 + rbase) * 128, nsub * 128)])

        if full:
            lax.fori_loop(0, full, lambda t, _: (do(t * 2, 2), 0)[1], 0)
        if rem:
            do(full * 2, rem)

    return k(tab_s, tab_d, rec)


def _edge_message(tab, st, ob, rec, ex):
    nrows = rec.shape[0]
    tile_rows = nrows // NS
    tile_rows_d = nrows // (NC * NS)

    @functools.partial(
        pl.kernel,
        out_type=(jax.ShapeDtypeStruct((HEADS, NPAD, HID), jnp.float32),
                  jax.ShapeDtypeStruct((NC, NPAD, 128), jnp.float32)),
        mesh=_sc_mesh(),
        scratch_types=[
            pltpu.VMEM((1, 2, 128), jnp.int32),
            pltpu.VMEM((1, 1, 128), jnp.int32),
            pltpu.VMEM((128, 16), jnp.float32),
            pltpu.VMEM((128, 128), jnp.float32),
            pltpu.VMEM_SHARED((SC_ROWS, HID), jnp.float32),
            pltpu.SemaphoreType.DMA,
        ])
    def k(tab_h, rec_h, ex_h, out_h, den_h,
          recv, gidx, exv, rows, out_sh, sem):
        c = lax.axis_index("c")
        s = lax.axis_index("s")

        def zero_rows():
            def fz(i, _):
                for t in range(8):
                    rows[i, pl.ds(t * 16, 16)] = jnp.zeros((16,), jnp.float32)
                return 0
            lax.fori_loop(0, 128, fz, 0)

        def zero_out_sh():
            zero_rows()
            for z in range(4):
                pltpu.sync_copy(rows,
                                out_sh.at[pl.ds(s * 632 + z * 128, 128)])
            pltpu.sync_copy(rows.at[pl.ds(0, 120)],
                            out_sh.at[pl.ds(s * 632 + 512, 120)])

        gdn = lax.GatherDimensionNumbers(
            offset_dims=(), collapsed_slice_dims=(0,), start_index_map=(0,))

        for h in range(4):
            head = c * 4 + h
            hidx = jnp.full((16, 1), head, jnp.int32)
            row0 = s * tile_rows
            zero_out_sh()
            plsc.subcore_barrier()

            def do(r, _):
                pltpu.sync_copy(rec_h.at[pl.ds(row0 + r, 1)], recv)
                for t in range(8):
                    gidx[0, 0, pl.ds(t * 16, 16)] = (
                        recv[0, 0, pl.ds(t * 16, 16)] * st + (ob + head))
                cp = pltpu.async_copy(tab_h.at[gidx.at[0, 0]], rows, sem)
                pltpu.sync_copy(ex_h.at[pl.ds((row0 + r) * 128, 128)], exv)
                cp.wait()

                def scale(e, _):
                    ev = exv[e, :]
                    wv = lax.gather(
                        ev, hidx, gdn, slice_sizes=(1,),
                        mode=lax.GatherScatterMode.PROMISE_IN_BOUNDS)
                    for t in range(8):
                        rows[e, pl.ds(t * 16, 16)] = (
                            rows[e, pl.ds(t * 16, 16)] * wv)
                    return 0
                lax.fori_loop(0, 128, scale, 0)
                pltpu.sync_copy(rows, out_sh.at[recv.at[0, 1]], add=True)
                return 0

            lax.fori_loop(0, tile_rows, do, 0)
            plsc.subcore_barrier()
            pltpu.sync_copy(out_sh.at[pl.ds(s * 632, 632)],
                            out_h.at[head, pl.ds(s * 632, 632)])
            plsc.subcore_barrier()

        zero_out_sh()
        plsc.subcore_barrier()
        row0_d = (s * NC + c) * tile_rows_d

        def do_den(r, _):
            pltpu.sync_copy(rec_h.at[pl.ds(row0_d + r, 1)], recv)
            pltpu.sync_copy(ex_h.at[pl.ds((row0_d + r) * 128, 128)], exv)

            def cpex(e, _):
                rows[e, pl.ds(0, 16)] = exv[e, :]
                return 0
            lax.fori_loop(0, 128, cpex, 0)
            pltpu.sync_copy(rows, out_sh.at[recv.at[0, 1]], add=True)
            return 0

        lax.fori_loop(0, tile_rows_d, do_den, 0)
        plsc.subcore_barrier()
        pltpu.sync_copy(out_sh.at[pl.ds(s * 632, 632)],
                        den_h.at[c, pl.ds(s * 632, 632)])

    return k(tab, rec, ex)


def _edge_final(ptab_s, ptab_d, rec, eap, w2):
    nrows = rec.shape[0]
    epad = nrows * 128
    tile_rows = nrows // (NC * NS)

    @functools.partial(
        pl.kernel,
        out_type=jax.ShapeDtypeStruct((epad, 16), jnp.float32),
        mesh=_sc_mesh(),
        scratch_types=[
            pltpu.VMEM((1, 2, 128), jnp.int32),
            pltpu.VMEM((1, 1, 128), jnp.int32),
            pltpu.VMEM((1, 1, 128), jnp.int32),
            pltpu.VMEM((128, 128), jnp.float32),
            pltpu.VMEM((128, 128), jnp.float32),
            pltpu.VMEM((128, 128), jnp.float32),
            pltpu.VMEM((128,), jnp.float32),
            pltpu.VMEM((128, 16), jnp.float32),
            pltpu.SemaphoreType.DMA,
        ])
    def k(ps_h, pd_h, rec_h, eap_h, w2_h, lanes_h,
          recv, gs, gd, psv, pdv, eav, w2v, lanev, sem):
        c = lax.axis_index("c")
        s = lax.axis_index("s")
        w = s * NC + c
        row0 = w * tile_rows
        pltpu.sync_copy(w2_h, w2v)

        def do(r, _):
            pltpu.sync_copy(rec_h.at[pl.ds(row0 + r, 1)], recv)
            for t in range(8):
                gs[0, 0, pl.ds(t * 16, 16)] = recv[0, 0, pl.ds(t * 16, 16)] * 2
                gd[0, 0, pl.ds(t * 16, 16)] = (
                    recv[0, 1, pl.ds(t * 16, 16)] * 2 + 1)
            cp1 = pltpu.async_copy(ps_h.at[gs.at[0, 0]], psv, sem)
            cp2 = pltpu.async_copy(pd_h.at[gd.at[0, 0]], pdv, sem)
            pltpu.sync_copy(eap_h.at[pl.ds((row0 + r) * 128, 128)], eav)
            cp1.wait()
            cp2.wait()

            def comp(e, _):
                u = jnp.zeros((16,), jnp.float32)
                for t in range(8):
                    v = (psv[e, pl.ds(t * 16, 16)]
                         + pdv[e, pl.ds(t * 16, 16)]
                         + eav[e, pl.ds(t * 16, 16)])
                    v = jnp.where(v > 0, v, 0.01 * v)
                    u = u + v * w2v[pl.ds(t * 16, 16)]
                lanev[e, :] = u
                return 0
            lax.fori_loop(0, 128, comp, 0)
            pltpu.sync_copy(lanev,
                            lanes_h.at[pl.ds((row0 + r) * 128, 128)])
            return 0

        lax.fori_loop(0, tile_rows, do, 0)

    return k(ptab_s, ptab_d, rec, eap, w2)


def _alcols(w, a):
    wr = w.reshape(w.shape[0], HEADS, HID)
    al = jnp.einsum("dhj,hj->dh", wr, a)
    return jnp.pad(al, ((0, 0), (0, 8)))


def _wcat(w, a):
    return jnp.concatenate(
        [w, _alcols(w, a), jnp.zeros((w.shape[0], MBLK - HC - 16), jnp.float32)],
        axis=1)


def _pad_rows(x, n):
    return jnp.pad(x, ((0, n - x.shape[0]), (0, 0)))


def kernel(x_request, x_vehicle,
           edge_index_rr, edge_attr_rr,
           edge_index_vr, edge_attr_vr,
           edge_index_rv, edge_attr_rv,
           Wsrc_rr1, Wdst_rr1, asrc_rr1, adst_rr1, b_rr1,
           Wsrc_vr1, Wdst_vr1, asrc_vr1, adst_vr1, b_vr1,
           Wsrc_rv1, Wdst_rv1, asrc_rv1, adst_rv1, b_rv1,
           Wsrc_rr2, Wdst_rr2, asrc_rr2, adst_rr2, b_rr2,
           Wsrc_vr2, Wdst_vr2, asrc_vr2, adst_vr2, b_vr2,
           Wsrc_rv2, Wdst_rv2, asrc_rv2, adst_rv2, b_rv2,
           Wep_rr, bep_rr, Wep_vr, bep_vr, Wep_rv, bep_rv,
           W1, b1, W2, b2):
    p = dict(locals())
    etypes = [("rr", "request", "request"),
              ("vr", "vehicle", "request"),
              ("rv", "request", "vehicle")]

    src = {}
    dst = {}
    for et in EPAD:
        ei = p[f"edge_index_{et}"]
        e = EREAL[et]
        s = jnp.full((EPAD[et],), 10000, jnp.int32).at[:e].set(ei[0])
        d = jnp.full((EPAD[et],), 10000, jnp.int32).at[:e].set(ei[1])
        src[et] = _pack_rec(s.reshape(-1, 128), d.reshape(-1, 128))

    x = {"request": _pad_rows(x_request, NPAD),
         "vehicle": _pad_rows(x_vehicle, NPAD)}

    for l in (1, 2):
        wc = {et: {"s": _wcat(p[f"Wsrc_{et}{l}"], p[f"asrc_{et}{l}"]),
                   "d": _wcat(p[f"Wdst_{et}{l}"], p[f"adst_{et}{l}"])}
              for et in EPAD}
        w_req = jnp.concatenate(
            [wc["rr"]["s"], wc["rr"]["d"], wc["vr"]["d"], wc["rv"]["s"]], axis=1)
        w_veh = jnp.concatenate([wc["vr"]["s"], wc["rv"]["d"]], axis=1)
        if l == 1:
            out_req = _mm_std(x["request"], w_req)
            out_veh = _mm_std(x["vehicle"], w_veh)
        else:
            out_req = _mm_hm(x["request"], w_req.reshape(HEADS, HID, -1))
            out_veh = _mm_hm(x["vehicle"], w_veh.reshape(HEADS, HID, -1))

        tab_req = out_req.reshape(-1, HID)
        tab_veh = out_veh.reshape(-1, HID)
        s_req, s_veh = 4 * 9, 2 * 9
        tcfg = {"rr": ((tab_req, s_req, 0), (tab_req, s_req, 1)),
                "vr": ((tab_veh, s_veh, 0), (tab_req, s_req, 2)),
                "rv": ((tab_req, s_req, 3), (tab_veh, s_veh, 1))}

        raws = {"request": [], "vehicle": []}
        dens = {"request": [], "vehicle": []}
        bs = {"request": [], "vehicle": []}
        for et, st, dt in etypes:
            (ts, ss, ibs), (td, sd, ibd) = tcfg[et]
            ex = _edge_exp(ts, ss, ibs * 9 + 8, td, sd, ibd * 9 + 8, src[et])
            raw, den = _edge_message(ts, ss, ibs * 9, src[et], ex)
            raws[dt].append(raw)
            dens[dt].append(den)
            bs[dt].append(p[f"b_{et}{l}"].reshape(HEADS, HID))

        newx = {}
        for dt in ("request", "vehicle"):
            resid = x[dt] if l == 2 else None
            newx[dt] = _combine(raws[dt], dens[dt], bs[dt], resid)
        x = newx

    W1s, W1e, W1t = W1[:HC], W1[HC:HC + HID], W1[HC + HID:]
    wst = jnp.concatenate([W1s.reshape(HEADS, HID, HID),
                           W1t.reshape(HEADS, HID, HID)], axis=2)
    ptab = {"request": _mm_hm(x["request"], wst).reshape(NPAD * 2, HID),
            "vehicle": _mm_hm(x["vehicle"], wst).reshape(NPAD * 2, HID)}
    w2 = W2.reshape(HID)

    outs = []
    for et, st, dt in etypes:
        wea = p[f"Wep_{et}"] @ W1e
        bea = p[f"bep_{et}"] @ W1e + b1
        ea = jnp.pad(p[f"edge_attr_{et}"], ((0, EPAD[et] - EREAL[et]), (0, 0)))
        eap = _mm_std(ea, wea, bea)
        lanes = _edge_final(ptab[st], ptab[dt], src[et], eap, w2)
        res = _lanesum(lanes, b2[0])
        outs.append(res[:EREAL[et]])
    return jnp.concatenate(outs, axis=0)
